# baseline (speedup 1.0000x reference)
import jax
import jax.numpy as jnp
import numpy as np
from jax.experimental import pallas as pl

B = 4096
FANOUT = 32
DN = 128
DE = 16
DT = 32
DEMB = 128
H = 8
DH = DEMB // H

BN = 64


def _f32dot(a, b):
    return jax.lax.dot(a.astype(jnp.bfloat16), b.astype(jnp.bfloat16),
                       preferred_element_type=jnp.float32)


_COS_C = [np.float32(c) for c in
          (0.9999992, -19.738981, 64.92866, -85.27162, 58.790497, -21.071106)]
_INV_2PI = np.float32(1.0 / (2.0 * np.pi))
_RND = np.float32(12582912.0)


def _fast_cos(x):
    y = x * _INV_2PI
    y = y - ((y + _RND) - _RND)
    t = y * y
    acc = _COS_C[5]
    for c in _COS_C[4::-1]:
        acc = acc * t + c
    return acc


def _fused_kernel(
    root_s, root_d, root_n,
    src_s, src_d, src_n,
    edge_s, edge_d, edge_n,
    dt_s, dt_d, dt_n,
    tw, tb,
    Wq_r, q_const,
    Wk_src, Wk_e, Wk_t, bk,
    Wv_src, Wv_e, Wv_t, bv,
    Wm1_a, Wm1_r, bm1, Wm2, bm2,
    Ws, Wd, b_sd, Wo, bo,
    hsum, hexp,
    pos_out, neg_out,
):
    def node_embed(root_ref, src_ref, edge_ref, dt_ref):
        M = BN
        R = M * FANOUT
        bf16 = jnp.bfloat16
        root_b = root_ref[...]
        src_b = src_ref[...]
        edge_b = edge_ref[...]
        dt = dt_ref[...]

        te = _fast_cos(dt * tw[...] + tb[...])
        te_b = te.astype(bf16)

        k = (_f32dot(src_b, Wk_src[...]) + _f32dot(edge_b, Wk_e[...])
             + _f32dot(te_b, Wk_t[...]) + bk[...])
        v = (_f32dot(src_b, Wv_src[...]) + _f32dot(edge_b, Wv_e[...])
             + _f32dot(te_b, Wv_t[...]) + bv[...])
        q = _f32dot(root_b, Wq_r[...]) + q_const[...]

        q_r = jnp.broadcast_to(q[:, None, :], (M, FANOUT, DEMB)).reshape(R, DEMB)
        e = jnp.exp(_f32dot((q_r * k).astype(bf16), hsum[...]))
        denom = jnp.sum(e.reshape(M, FANOUT, H), axis=1)
        e_exp = _f32dot(e.astype(bf16), hexp[...])
        num = jnp.sum((e_exp * v).reshape(M, FANOUT, DEMB), axis=1)
        agg = num / _f32dot(denom.astype(bf16), hexp[...])

        hmid = jnp.maximum(
            _f32dot(agg.astype(bf16), Wm1_a[...])
            + _f32dot(root_b, Wm1_r[...]) + bm1[...],
            0.0)
        return _f32dot(hmid.astype(bf16), Wm2[...]) + bm2[...]

    hs = node_embed(root_s, src_s, edge_s, dt_s)
    hd = node_embed(root_d, src_d, edge_d, dt_d)
    hn = node_embed(root_n, src_n, edge_n, dt_n)
    sc = _f32dot(hs, Ws[...]) + b_sd[...]
    pos = _f32dot(jnp.maximum(sc + _f32dot(hd, Wd[...]), 0.0), Wo[...]) + bo[...]
    neg = _f32dot(jnp.maximum(sc + _f32dot(hn, Wd[...]), 0.0), Wo[...]) + bo[...]
    pos_out[...] = pos
    neg_out[...] = neg


def kernel(root_feat, src_feat, edge_feat, delta_time, time_w, time_b,
           Wq, bq, Wk, bk, Wv, bv, Wm1, bm1, Wm2, bm2,
           Ws, bs, Wd, bd, Wo, bo):
    nb = B // BN
    f32 = jnp.float32

    bf16 = jnp.bfloat16
    n_total = src_feat.shape[0]
    root_bf = root_feat.astype(bf16)
    src2 = src_feat.reshape(n_total * FANOUT, DN).astype(bf16)
    edge2 = edge_feat.reshape(n_total * FANOUT, DE).astype(bf16)
    dt2 = delta_time.reshape(n_total * FANOUT, 1)

    tw = time_w.reshape(1, DT)
    tb = time_b.reshape(1, DT)
    q_const = jnp.cos(time_b).reshape(1, DT) @ Wq[DN:] + bq.reshape(1, DEMB)
    Wq_r = Wq[:DN]
    Wk_src, Wk_e, Wk_t = Wk[:DN], Wk[DN:DN + DE], Wk[DN + DE:]
    Wv_src, Wv_e, Wv_t = Wv[:DN], Wv[DN:DN + DE], Wv[DN + DE:]
    Wm1_a, Wm1_r = Wm1[:DEMB], Wm1[DEMB:]
    b_sd = (bs + bd).reshape(1, DEMB)

    lane = jnp.arange(DEMB)[:, None] // DH
    head = jnp.arange(H)[None, :]
    hexp = (lane == head).astype(f32).T
    hsum = hexp.T * np.float32(1.0 / np.sqrt(DH))

    def rep3(shape, extra_dims):
        zeros = (0,) * extra_dims
        return [
            pl.BlockSpec(shape, lambda i: (i,) + zeros),
            pl.BlockSpec(shape, lambda i: (i + nb,) + zeros),
            pl.BlockSpec(shape, lambda i: (i + 2 * nb,) + zeros),
        ]

    def full(arr):
        nd = arr.ndim
        return pl.BlockSpec(arr.shape, lambda i: (0,) * nd)

    def wb(w):
        return w.astype(bf16)

    weights = [tw, tb, wb(Wq_r), q_const,
               wb(Wk_src), wb(Wk_e), wb(Wk_t), bk.reshape(1, DEMB),
               wb(Wv_src), wb(Wv_e), wb(Wv_t), bv.reshape(1, DEMB),
               wb(Wm1_a), wb(Wm1_r), bm1.reshape(1, DEMB),
               wb(Wm2), bm2.reshape(1, DEMB),
               wb(Ws), wb(Wd), b_sd, wb(Wo), bo.reshape(1, 1),
               wb(hsum), wb(hexp)]

    in_specs = (rep3((BN, DN), 1)
                + rep3((BN * FANOUT, DN), 1)
                + rep3((BN * FANOUT, DE), 1)
                + rep3((BN * FANOUT, 1), 1)
                + [full(w) for w in weights])

    out_shape = (jax.ShapeDtypeStruct((B, 1), f32),
                 jax.ShapeDtypeStruct((B, 1), f32))
    out_specs = (pl.BlockSpec((BN, 1), lambda i: (i, 0)),
                 pl.BlockSpec((BN, 1), lambda i: (i, 0)))

    pos, neg = pl.pallas_call(
        _fused_kernel,
        grid=(nb,),
        in_specs=in_specs,
        out_specs=out_specs,
        out_shape=out_shape,
    )(root_bf, root_bf, root_bf,
      src2, src2, src2,
      edge2, edge2, edge2,
      dt2, dt2, dt2,
      *weights)

    return pos[:, 0], neg[:, 0]


# speedup vs baseline: 1.1331x; 1.1331x over previous
import jax
import jax.numpy as jnp
import numpy as np
from jax.experimental import pallas as pl

B = 4096
FANOUT = 32
DN = 128
DE = 16
DT = 32
DEMB = 128
H = 8
DH = DEMB // H

BN = 64


def _f32dot(a, b):
    return jax.lax.dot(a.astype(jnp.bfloat16), b.astype(jnp.bfloat16),
                       preferred_element_type=jnp.float32)


_COS_C = [np.float32(c) for c in
          (0.9999992, -19.738981, 64.92866, -85.27162, 58.790497, -21.071106)]
_INV_2PI = np.float32(1.0 / (2.0 * np.pi))
_RND = np.float32(12582912.0)


def _fast_cos(x):
    y = x * _INV_2PI
    y = y - ((y + _RND) - _RND)
    t = y * y
    acc = _COS_C[5]
    for c in _COS_C[4::-1]:
        acc = acc * t + c
    return acc


def _fused_kernel(
    root_s, root_d, root_n,
    src_s, src_d, src_n,
    edge_s, edge_d, edge_n,
    dt_s, dt_d, dt_n,
    tw, tb,
    Wq_r, q_const,
    Wk_src, Wk_e, Wk_t, bk,
    Wv_src, Wv_e, Wv_t, bv,
    Wm1_a, Wm1_r, bm1, Wm2, bm2,
    Ws, Wd, b_sd, Wo, bo,
    hsum, hexp,
    pos_out, neg_out,
):
    def node_embed(root_ref, src_ref, edge_ref, dt_ref):
        M = BN
        R = M * FANOUT
        bf16 = jnp.bfloat16
        root_b = root_ref[...]
        src_b = src_ref[...].astype(bf16).reshape(R, DN)
        edge_b = edge_ref[...]
        dt = dt_ref[...]

        te = _fast_cos(dt * tw[...] + tb[...])
        te_b = te.astype(bf16)

        k = (_f32dot(src_b, Wk_src[...]) + _f32dot(edge_b, Wk_e[...])
             + _f32dot(te_b, Wk_t[...]) + bk[...])
        v = (_f32dot(src_b, Wv_src[...]) + _f32dot(edge_b, Wv_e[...])
             + _f32dot(te_b, Wv_t[...]) + bv[...])
        q = _f32dot(root_b, Wq_r[...]) + q_const[...]

        q_r = jnp.broadcast_to(q[:, None, :], (M, FANOUT, DEMB)).reshape(R, DEMB)
        e = jnp.exp(_f32dot((q_r * k).astype(bf16), hsum[...]))
        denom = jnp.sum(e.reshape(M, FANOUT, H), axis=1)
        e_exp = _f32dot(e.astype(bf16), hexp[...])
        num = jnp.sum((e_exp * v).reshape(M, FANOUT, DEMB), axis=1)
        agg = num / _f32dot(denom.astype(bf16), hexp[...])

        hmid = jnp.maximum(
            _f32dot(agg.astype(bf16), Wm1_a[...])
            + _f32dot(root_b, Wm1_r[...]) + bm1[...],
            0.0)
        return _f32dot(hmid.astype(bf16), Wm2[...]) + bm2[...]

    hs = node_embed(root_s, src_s, edge_s, dt_s)
    hd = node_embed(root_d, src_d, edge_d, dt_d)
    hn = node_embed(root_n, src_n, edge_n, dt_n)
    sc = _f32dot(hs, Ws[...]) + b_sd[...]
    pos = _f32dot(jnp.maximum(sc + _f32dot(hd, Wd[...]), 0.0), Wo[...]) + bo[...]
    neg = _f32dot(jnp.maximum(sc + _f32dot(hn, Wd[...]), 0.0), Wo[...]) + bo[...]
    pos_out[...] = pos
    neg_out[...] = neg


def kernel(root_feat, src_feat, edge_feat, delta_time, time_w, time_b,
           Wq, bq, Wk, bk, Wv, bv, Wm1, bm1, Wm2, bm2,
           Ws, bs, Wd, bd, Wo, bo):
    nb = B // BN
    f32 = jnp.float32

    bf16 = jnp.bfloat16
    n_total = src_feat.shape[0]
    root_bf = root_feat.astype(bf16)
    edge2 = edge_feat.reshape(n_total * FANOUT, DE).astype(bf16)
    dt2 = delta_time.reshape(n_total * FANOUT, 1)

    tw = time_w.reshape(1, DT)
    tb = time_b.reshape(1, DT)
    q_const = jnp.cos(time_b).reshape(1, DT) @ Wq[DN:] + bq.reshape(1, DEMB)
    Wq_r = Wq[:DN]
    Wk_src, Wk_e, Wk_t = Wk[:DN], Wk[DN:DN + DE], Wk[DN + DE:]
    Wv_src, Wv_e, Wv_t = Wv[:DN], Wv[DN:DN + DE], Wv[DN + DE:]
    Wm1_a, Wm1_r = Wm1[:DEMB], Wm1[DEMB:]
    b_sd = (bs + bd).reshape(1, DEMB)

    lane = jnp.arange(DEMB)[:, None] // DH
    head = jnp.arange(H)[None, :]
    hexp = (lane == head).astype(f32).T
    hsum = hexp.T * np.float32(1.0 / np.sqrt(DH))

    def rep3(shape, extra_dims):
        zeros = (0,) * extra_dims
        return [
            pl.BlockSpec(shape, lambda i: (i,) + zeros),
            pl.BlockSpec(shape, lambda i: (i + nb,) + zeros),
            pl.BlockSpec(shape, lambda i: (i + 2 * nb,) + zeros),
        ]

    def full(arr):
        nd = arr.ndim
        return pl.BlockSpec(arr.shape, lambda i: (0,) * nd)

    def wb(w):
        return w.astype(bf16)

    weights = [tw, tb, wb(Wq_r), q_const,
               wb(Wk_src), wb(Wk_e), wb(Wk_t), bk.reshape(1, DEMB),
               wb(Wv_src), wb(Wv_e), wb(Wv_t), bv.reshape(1, DEMB),
               wb(Wm1_a), wb(Wm1_r), bm1.reshape(1, DEMB),
               wb(Wm2), bm2.reshape(1, DEMB),
               wb(Ws), wb(Wd), b_sd, wb(Wo), bo.reshape(1, 1),
               wb(hsum), wb(hexp)]

    in_specs = (rep3((BN, DN), 1)
                + rep3((BN, FANOUT, DN), 2)
                + rep3((BN * FANOUT, DE), 1)
                + rep3((BN * FANOUT, 1), 1)
                + [full(w) for w in weights])

    out_shape = (jax.ShapeDtypeStruct((B, 1), f32),
                 jax.ShapeDtypeStruct((B, 1), f32))
    out_specs = (pl.BlockSpec((BN, 1), lambda i: (i, 0)),
                 pl.BlockSpec((BN, 1), lambda i: (i, 0)))

    pos, neg = pl.pallas_call(
        _fused_kernel,
        grid=(nb,),
        in_specs=in_specs,
        out_specs=out_specs,
        out_shape=out_shape,
    )(root_bf, root_bf, root_bf,
      src_feat, src_feat, src_feat,
      edge2, edge2, edge2,
      dt2, dt2, dt2,
      *weights)

    return pos[:, 0], neg[:, 0]


# speedup vs baseline: 1.1920x; 1.0519x over previous
import jax
import jax.numpy as jnp
import numpy as np
from jax.experimental import pallas as pl

B = 4096
FANOUT = 32
DN = 128
DE = 16
DT = 32
DEMB = 128
H = 8
DH = DEMB // H

BN = 128


def _f32dot(a, b):
    return jax.lax.dot(a.astype(jnp.bfloat16), b.astype(jnp.bfloat16),
                       preferred_element_type=jnp.float32)


_COS_C = [np.float32(c) for c in
          (0.9999992, -19.738981, 64.92866, -85.27162, 58.790497, -21.071106)]
_INV_2PI = np.float32(1.0 / (2.0 * np.pi))
_RND = np.float32(12582912.0)


def _fast_cos(x):
    y = x * _INV_2PI
    y = y - ((y + _RND) - _RND)
    t = y * y
    acc = _COS_C[5]
    for c in _COS_C[4::-1]:
        acc = acc * t + c
    return acc


def _fused_kernel(
    root_s, root_d, root_n,
    src_s, src_d, src_n,
    edge_s, edge_d, edge_n,
    dt_s, dt_d, dt_n,
    tw, tb,
    Wq_r, q_const,
    Wk_src, Wk_e, Wk_t, bk,
    Wv_src, Wv_e, Wv_t, bv,
    Wm1_a, Wm1_r, bm1, Wm2, bm2,
    Ws, Wd, b_sd, Wo, bo,
    hsum, hexp,
    pos_out, neg_out,
):
    def node_embed(root_ref, src_ref, edge_ref, dt_ref):
        M = BN
        R = M * FANOUT
        bf16 = jnp.bfloat16
        root_b = root_ref[...]
        src_b = src_ref[...].astype(bf16).reshape(R, DN)
        edge_b = edge_ref[...]
        dt = dt_ref[...]

        te = _fast_cos(dt * tw[...] + tb[...])
        te_b = te.astype(bf16)

        k = (_f32dot(src_b, Wk_src[...]) + _f32dot(edge_b, Wk_e[...])
             + _f32dot(te_b, Wk_t[...]) + bk[...])
        v = (_f32dot(src_b, Wv_src[...]) + _f32dot(edge_b, Wv_e[...])
             + _f32dot(te_b, Wv_t[...]) + bv[...])
        q = _f32dot(root_b, Wq_r[...]) + q_const[...]

        q_r = jnp.broadcast_to(q[:, None, :], (M, FANOUT, DEMB)).reshape(R, DEMB)
        e = jnp.exp(_f32dot((q_r * k).astype(bf16), hsum[...]))
        denom = jnp.sum(e.reshape(M, FANOUT, H), axis=1)
        e_exp = _f32dot(e.astype(bf16), hexp[...])
        num = jnp.sum((e_exp * v).reshape(M, FANOUT, DEMB), axis=1)
        agg = num / _f32dot(denom.astype(bf16), hexp[...])

        hmid = jnp.maximum(
            _f32dot(agg.astype(bf16), Wm1_a[...])
            + _f32dot(root_b, Wm1_r[...]) + bm1[...],
            0.0)
        return _f32dot(hmid.astype(bf16), Wm2[...]) + bm2[...]

    hs = node_embed(root_s, src_s, edge_s, dt_s)
    hd = node_embed(root_d, src_d, edge_d, dt_d)
    hn = node_embed(root_n, src_n, edge_n, dt_n)
    sc = _f32dot(hs, Ws[...]) + b_sd[...]
    pos = _f32dot(jnp.maximum(sc + _f32dot(hd, Wd[...]), 0.0), Wo[...]) + bo[...]
    neg = _f32dot(jnp.maximum(sc + _f32dot(hn, Wd[...]), 0.0), Wo[...]) + bo[...]
    pos_out[...] = pos
    neg_out[...] = neg


def kernel(root_feat, src_feat, edge_feat, delta_time, time_w, time_b,
           Wq, bq, Wk, bk, Wv, bv, Wm1, bm1, Wm2, bm2,
           Ws, bs, Wd, bd, Wo, bo):
    nb = B // BN
    f32 = jnp.float32

    bf16 = jnp.bfloat16
    n_total = src_feat.shape[0]
    root_bf = root_feat.astype(bf16)
    edge2 = edge_feat.reshape(n_total * FANOUT, DE).astype(bf16)
    dt2 = delta_time.reshape(n_total * FANOUT, 1)

    tw = time_w.reshape(1, DT)
    tb = time_b.reshape(1, DT)
    q_const = jnp.cos(time_b).reshape(1, DT) @ Wq[DN:] + bq.reshape(1, DEMB)
    Wq_r = Wq[:DN]
    Wk_src, Wk_e, Wk_t = Wk[:DN], Wk[DN:DN + DE], Wk[DN + DE:]
    Wv_src, Wv_e, Wv_t = Wv[:DN], Wv[DN:DN + DE], Wv[DN + DE:]
    Wm1_a, Wm1_r = Wm1[:DEMB], Wm1[DEMB:]
    b_sd = (bs + bd).reshape(1, DEMB)

    lane = jnp.arange(DEMB)[:, None] // DH
    head = jnp.arange(H)[None, :]
    hexp = (lane == head).astype(f32).T
    hsum = hexp.T * np.float32(1.0 / np.sqrt(DH))

    def rep3(shape, extra_dims):
        zeros = (0,) * extra_dims
        return [
            pl.BlockSpec(shape, lambda i: (i,) + zeros),
            pl.BlockSpec(shape, lambda i: (i + nb,) + zeros),
            pl.BlockSpec(shape, lambda i: (i + 2 * nb,) + zeros),
        ]

    def full(arr):
        nd = arr.ndim
        return pl.BlockSpec(arr.shape, lambda i: (0,) * nd)

    def wb(w):
        return w.astype(bf16)

    weights = [tw, tb, wb(Wq_r), q_const,
               wb(Wk_src), wb(Wk_e), wb(Wk_t), bk.reshape(1, DEMB),
               wb(Wv_src), wb(Wv_e), wb(Wv_t), bv.reshape(1, DEMB),
               wb(Wm1_a), wb(Wm1_r), bm1.reshape(1, DEMB),
               wb(Wm2), bm2.reshape(1, DEMB),
               wb(Ws), wb(Wd), b_sd, wb(Wo), bo.reshape(1, 1),
               wb(hsum), wb(hexp)]

    in_specs = (rep3((BN, DN), 1)
                + rep3((BN, FANOUT, DN), 2)
                + rep3((BN * FANOUT, DE), 1)
                + rep3((BN * FANOUT, 1), 1)
                + [full(w) for w in weights])

    out_shape = (jax.ShapeDtypeStruct((B, 1), f32),
                 jax.ShapeDtypeStruct((B, 1), f32))
    out_specs = (pl.BlockSpec((BN, 1), lambda i: (i, 0)),
                 pl.BlockSpec((BN, 1), lambda i: (i, 0)))

    pos, neg = pl.pallas_call(
        _fused_kernel,
        grid=(nb,),
        in_specs=in_specs,
        out_specs=out_specs,
        out_shape=out_shape,
    )(root_bf, root_bf, root_bf,
      src_feat, src_feat, src_feat,
      edge2, edge2, edge2,
      dt2, dt2, dt2,
      *weights)

    return pos[:, 0], neg[:, 0]


# speedup vs baseline: 1.1936x; 1.0013x over previous
import jax
import jax.numpy as jnp
import numpy as np
from jax.experimental import pallas as pl
from jax.experimental.pallas import tpu as pltpu

B = 4096
FANOUT = 32
DN = 128
DE = 16
DT = 32
DEMB = 128
H = 8
DH = DEMB // H

BN = 128


def _f32dot(a, b):
    return jax.lax.dot(a.astype(jnp.bfloat16), b.astype(jnp.bfloat16),
                       preferred_element_type=jnp.float32)


_COS_C = [np.float32(c) for c in
          (0.9999992, -19.738981, 64.92866, -85.27162, 58.790497, -21.071106)]
_INV_2PI = np.float32(1.0 / (2.0 * np.pi))
_RND = np.float32(12582912.0)


def _fast_cos(x):
    y = x * _INV_2PI
    y = y - ((y + _RND) - _RND)
    t = y * y
    acc = _COS_C[5]
    for c in _COS_C[4::-1]:
        acc = acc * t + c
    return acc


def _fused_kernel(
    root_s, root_d, root_n,
    src_s, src_d, src_n,
    edge_s, edge_d, edge_n,
    dt_s, dt_d, dt_n,
    tw, tb,
    Wq_r, q_const,
    Wk_src, Wk_e, Wk_t, bk,
    Wv_src, Wv_e, Wv_t, bv,
    Wm1_a, Wm1_r, bm1, Wm2, bm2,
    Ws, Wd, b_sd, Wo, bo,
    hsum, hexp,
    pos_out, neg_out,
):
    def node_embed(root_ref, src_ref, edge_ref, dt_ref):
        M = BN
        R = M * FANOUT
        bf16 = jnp.bfloat16
        root_b = root_ref[...]
        src_b = src_ref[...].astype(bf16).reshape(R, DN)
        edge_b = edge_ref[...]
        dt = dt_ref[...]

        te = _fast_cos(dt * tw[...] + tb[...])
        te_b = te.astype(bf16)

        k = (_f32dot(src_b, Wk_src[...]) + _f32dot(edge_b, Wk_e[...])
             + _f32dot(te_b, Wk_t[...]) + bk[...])
        v = (_f32dot(src_b, Wv_src[...]) + _f32dot(edge_b, Wv_e[...])
             + _f32dot(te_b, Wv_t[...]) + bv[...])
        q = _f32dot(root_b, Wq_r[...]) + q_const[...]

        q_r = jnp.broadcast_to(q[:, None, :], (M, FANOUT, DEMB)).reshape(R, DEMB)
        e = jnp.exp(_f32dot((q_r * k).astype(bf16), hsum[...]))
        denom = jnp.sum(e.reshape(M, FANOUT, H), axis=1)
        e_exp = _f32dot(e.astype(bf16), hexp[...])
        num = jnp.sum((e_exp * v).reshape(M, FANOUT, DEMB), axis=1)
        agg = num / _f32dot(denom.astype(bf16), hexp[...])

        hmid = jnp.maximum(
            _f32dot(agg.astype(bf16), Wm1_a[...])
            + _f32dot(root_b, Wm1_r[...]) + bm1[...],
            0.0)
        return _f32dot(hmid.astype(bf16), Wm2[...]) + bm2[...]

    hs = node_embed(root_s, src_s, edge_s, dt_s)
    hd = node_embed(root_d, src_d, edge_d, dt_d)
    hn = node_embed(root_n, src_n, edge_n, dt_n)
    sc = _f32dot(hs, Ws[...]) + b_sd[...]
    pos = _f32dot(jnp.maximum(sc + _f32dot(hd, Wd[...]), 0.0), Wo[...]) + bo[...]
    neg = _f32dot(jnp.maximum(sc + _f32dot(hn, Wd[...]), 0.0), Wo[...]) + bo[...]
    pos_out[...] = pos
    neg_out[...] = neg


def kernel(root_feat, src_feat, edge_feat, delta_time, time_w, time_b,
           Wq, bq, Wk, bk, Wv, bv, Wm1, bm1, Wm2, bm2,
           Ws, bs, Wd, bd, Wo, bo):
    nb = B // BN
    f32 = jnp.float32

    bf16 = jnp.bfloat16
    n_total = src_feat.shape[0]
    root_bf = root_feat.astype(bf16)
    edge2 = edge_feat.reshape(n_total * FANOUT, DE).astype(bf16)
    dt2 = delta_time.reshape(n_total * FANOUT, 1)

    tw = time_w.reshape(1, DT)
    tb = time_b.reshape(1, DT)
    q_const = jnp.cos(time_b).reshape(1, DT) @ Wq[DN:] + bq.reshape(1, DEMB)
    Wq_r = Wq[:DN]
    Wk_src, Wk_e, Wk_t = Wk[:DN], Wk[DN:DN + DE], Wk[DN + DE:]
    Wv_src, Wv_e, Wv_t = Wv[:DN], Wv[DN:DN + DE], Wv[DN + DE:]
    Wm1_a, Wm1_r = Wm1[:DEMB], Wm1[DEMB:]
    b_sd = (bs + bd).reshape(1, DEMB)

    lane = jnp.arange(DEMB)[:, None] // DH
    head = jnp.arange(H)[None, :]
    hexp = (lane == head).astype(f32).T
    hsum = hexp.T * np.float32(1.0 / np.sqrt(DH))

    def rep3(shape, extra_dims):
        zeros = (0,) * extra_dims
        return [
            pl.BlockSpec(shape, lambda i: (i,) + zeros),
            pl.BlockSpec(shape, lambda i: (i + nb,) + zeros),
            pl.BlockSpec(shape, lambda i: (i + 2 * nb,) + zeros),
        ]

    def full(arr):
        nd = arr.ndim
        return pl.BlockSpec(arr.shape, lambda i: (0,) * nd)

    def wb(w):
        return w.astype(bf16)

    weights = [tw, tb, wb(Wq_r), q_const,
               wb(Wk_src), wb(Wk_e), wb(Wk_t), bk.reshape(1, DEMB),
               wb(Wv_src), wb(Wv_e), wb(Wv_t), bv.reshape(1, DEMB),
               wb(Wm1_a), wb(Wm1_r), bm1.reshape(1, DEMB),
               wb(Wm2), bm2.reshape(1, DEMB),
               wb(Ws), wb(Wd), b_sd, wb(Wo), bo.reshape(1, 1),
               wb(hsum), wb(hexp)]

    in_specs = (rep3((BN, DN), 1)
                + rep3((BN, FANOUT, DN), 2)
                + rep3((BN * FANOUT, DE), 1)
                + rep3((BN * FANOUT, 1), 1)
                + [full(w) for w in weights])

    out_shape = (jax.ShapeDtypeStruct((B, 1), f32),
                 jax.ShapeDtypeStruct((B, 1), f32))
    out_specs = (pl.BlockSpec((BN, 1), lambda i: (i, 0)),
                 pl.BlockSpec((BN, 1), lambda i: (i, 0)))

    pos, neg = pl.pallas_call(
        _fused_kernel,
        grid=(nb,),
        in_specs=in_specs,
        out_specs=out_specs,
        out_shape=out_shape,
        compiler_params=pltpu.CompilerParams(
            dimension_semantics=("parallel",)),
    )(root_bf, root_bf, root_bf,
      src_feat, src_feat, src_feat,
      edge2, edge2, edge2,
      dt2, dt2, dt2,
      *weights)

    return pos[:, 0], neg[:, 0]


# speedup vs baseline: 1.7438x; 1.4610x over previous
import jax
import jax.numpy as jnp
import numpy as np
from jax.experimental import pallas as pl
from jax.experimental.pallas import tpu as pltpu

B = 4096
FANOUT = 32
DN = 128
DE = 16
DT = 32
DEMB = 128
H = 8
DH = DEMB // H

BN = 128


def _f32dot(a, b):
    return jax.lax.dot(a.astype(jnp.bfloat16), b.astype(jnp.bfloat16),
                       preferred_element_type=jnp.float32)


_COS_C = [np.float32(c) for c in
          (0.9999992, -19.738981, 64.92866, -85.27162, 58.790497, -21.071106)]
_INV_2PI = np.float32(1.0 / (2.0 * np.pi))
_RND = np.float32(12582912.0)


def _fast_cos(x):
    y = x * _INV_2PI
    y = y - ((y + _RND) - _RND)
    t = y * y
    acc = _COS_C[5]
    for c in _COS_C[4::-1]:
        acc = acc * t + c
    return acc


def _fused_kernel(
    root_s, root_d, root_n,
    src_s, src_d, src_n,
    edge_s, edge_d, edge_n,
    dt_s, dt_d, dt_n,
    tw, tb, fmask, ones32,
    Wq_r, q_const,
    Wk_src, Wk_e, Wk_t, bk,
    Wv_src, Wv_e, Wv_t, bv,
    Wm1_a, Wm1_r, bm1, Wm2, bm2,
    Ws, Wd, b_sd, Wo, bo,
    hsum, hexp,
    pos_out, neg_out,
):
    def node_embed(root_ref, src_ref, edge_ref, dt_ref):
        M = BN
        R = M * FANOUT
        bf16 = jnp.bfloat16
        root_b = root_ref[...]
        src_b = src_ref[...].astype(bf16).reshape(R, DN)
        edge_b = edge_ref[...]
        dt = dt_ref[...]

        dt_rows = jnp.broadcast_to(
            dt[:, None, :], (M, FANOUT, FANOUT)).reshape(R, FANOUT)
        dt_rep = jax.lax.dot(dt_rows * fmask[...], ones32[...],
                             preferred_element_type=jnp.float32)
        te = _fast_cos(dt_rep * tw[...] + tb[...])
        te_b = te.astype(bf16)

        k = (_f32dot(src_b, Wk_src[...]) + _f32dot(edge_b, Wk_e[...])
             + _f32dot(te_b, Wk_t[...]) + bk[...])
        v = (_f32dot(src_b, Wv_src[...]) + _f32dot(edge_b, Wv_e[...])
             + _f32dot(te_b, Wv_t[...]) + bv[...])
        q = _f32dot(root_b, Wq_r[...]) + q_const[...]

        q_r = jnp.broadcast_to(q[:, None, :], (M, FANOUT, DEMB)).reshape(R, DEMB)
        e = jnp.exp(_f32dot((q_r * k).astype(bf16), hsum[...]))
        denom = jnp.sum(e.reshape(M, FANOUT, H), axis=1)
        e_exp = _f32dot(e.astype(bf16), hexp[...])
        num = jnp.sum((e_exp * v).reshape(M, FANOUT, DEMB), axis=1)
        agg = num / _f32dot(denom.astype(bf16), hexp[...])

        hmid = jnp.maximum(
            _f32dot(agg.astype(bf16), Wm1_a[...])
            + _f32dot(root_b, Wm1_r[...]) + bm1[...],
            0.0)
        return _f32dot(hmid.astype(bf16), Wm2[...]) + bm2[...]

    hs = node_embed(root_s, src_s, edge_s, dt_s)
    hd = node_embed(root_d, src_d, edge_d, dt_d)
    hn = node_embed(root_n, src_n, edge_n, dt_n)
    sc = _f32dot(hs, Ws[...]) + b_sd[...]
    pos = _f32dot(jnp.maximum(sc + _f32dot(hd, Wd[...]), 0.0), Wo[...]) + bo[...]
    neg = _f32dot(jnp.maximum(sc + _f32dot(hn, Wd[...]), 0.0), Wo[...]) + bo[...]
    pos_out[...] = pos
    neg_out[...] = neg


def kernel(root_feat, src_feat, edge_feat, delta_time, time_w, time_b,
           Wq, bq, Wk, bk, Wv, bv, Wm1, bm1, Wm2, bm2,
           Ws, bs, Wd, bd, Wo, bo):
    nb = B // BN
    f32 = jnp.float32

    bf16 = jnp.bfloat16
    n_total = src_feat.shape[0]
    root_bf = root_feat.astype(bf16)
    edge2 = edge_feat.reshape(n_total * FANOUT, DE).astype(bf16)

    tw = time_w.reshape(1, DT)
    tb = time_b.reshape(1, DT)
    q_const = jnp.cos(time_b).reshape(1, DT) @ Wq[DN:] + bq.reshape(1, DEMB)
    Wq_r = Wq[:DN]
    Wk_src, Wk_e, Wk_t = Wk[:DN], Wk[DN:DN + DE], Wk[DN + DE:]
    Wv_src, Wv_e, Wv_t = Wv[:DN], Wv[DN:DN + DE], Wv[DN + DE:]
    Wm1_a, Wm1_r = Wm1[:DEMB], Wm1[DEMB:]
    b_sd = (bs + bd).reshape(1, DEMB)

    lane = jnp.arange(DEMB)[:, None] // DH
    head = jnp.arange(H)[None, :]
    hexp = (lane == head).astype(f32).T
    hsum = hexp.T * np.float32(1.0 / np.sqrt(DH))

    rr = jnp.arange(BN * FANOUT)[:, None] % FANOUT
    fmask = (rr == jnp.arange(FANOUT)[None, :]).astype(f32)
    ones32 = jnp.ones((FANOUT, DT), f32)

    def rep3(shape, extra_dims):
        zeros = (0,) * extra_dims
        return [
            pl.BlockSpec(shape, lambda i: (i,) + zeros),
            pl.BlockSpec(shape, lambda i: (i + nb,) + zeros),
            pl.BlockSpec(shape, lambda i: (i + 2 * nb,) + zeros),
        ]

    def full(arr):
        nd = arr.ndim
        return pl.BlockSpec(arr.shape, lambda i: (0,) * nd)

    def wb(w):
        return w.astype(bf16)

    weights = [tw, tb, fmask, ones32, wb(Wq_r), q_const,
               wb(Wk_src), wb(Wk_e), wb(Wk_t), bk.reshape(1, DEMB),
               wb(Wv_src), wb(Wv_e), wb(Wv_t), bv.reshape(1, DEMB),
               wb(Wm1_a), wb(Wm1_r), bm1.reshape(1, DEMB),
               wb(Wm2), bm2.reshape(1, DEMB),
               wb(Ws), wb(Wd), b_sd, wb(Wo), bo.reshape(1, 1),
               wb(hsum), wb(hexp)]

    in_specs = (rep3((BN, DN), 1)
                + rep3((BN, FANOUT, DN), 2)
                + rep3((BN * FANOUT, DE), 1)
                + rep3((BN, FANOUT), 1)
                + [full(w) for w in weights])

    out_shape = (jax.ShapeDtypeStruct((B, 1), f32),
                 jax.ShapeDtypeStruct((B, 1), f32))
    out_specs = (pl.BlockSpec((BN, 1), lambda i: (i, 0)),
                 pl.BlockSpec((BN, 1), lambda i: (i, 0)))

    pos, neg = pl.pallas_call(
        _fused_kernel,
        grid=(nb,),
        in_specs=in_specs,
        out_specs=out_specs,
        out_shape=out_shape,
        compiler_params=pltpu.CompilerParams(
            dimension_semantics=("parallel",)),
    )(root_bf, root_bf, root_bf,
      src_feat, src_feat, src_feat,
      edge2, edge2, edge2,
      delta_time, delta_time, delta_time,
      *weights)

    return pos[:, 0], neg[:, 0]
